# baseline (speedup 1.0000x reference)
import functools

import jax
import jax.numpy as jnp
from jax import lax
from jax.experimental import pallas as pl
from jax.experimental.pallas import tpu as pltpu
from jax.experimental.pallas import tpu_sc as plsc

B, N, D = 1024, 77, 768
NUM_NAMES = 1000
PLACEHOLDER_TOKEN = 265

_NC, _NS = 2, 16
_NW = _NC * _NS
_B_PER_W = B // _NW


def _sc_gather_body(name_hbm, params_hbm, out_hbm, idx_v, rows_v, sem):
    wid = lax.axis_index("s") * _NC + lax.axis_index("c")
    base = wid * _B_PER_W
    pltpu.sync_copy(name_hbm.at[pl.ds(base, _B_PER_W)], idx_v)
    pltpu.async_copy(params_hbm.at[idx_v], rows_v, sem).wait()
    pltpu.sync_copy(rows_v, out_hbm.at[pl.ds(base, _B_PER_W)])


@functools.cache
def _sc_gather():
    return pl.kernel(
        _sc_gather_body,
        out_type=jax.ShapeDtypeStruct((B, D), jnp.float32),
        mesh=plsc.VectorSubcoreMesh(core_axis_name="c", subcore_axis_name="s"),
        scratch_types=[
            pltpu.VMEM((_B_PER_W,), jnp.int32),
            pltpu.VMEM((_B_PER_W, D), jnp.float32),
            pltpu.SemaphoreType.DMA,
        ],
    )


def _merge_body(tok_ref, emb_ref, g_ref, out_ref):
    mask = tok_ref[...] == PLACEHOLDER_TOKEN
    out_ref[...] = jnp.where(mask, g_ref[...], emb_ref[...])


def _merge(tokenized_text, embedded_text, gathered, bb):
    tok3 = tokenized_text.reshape(B, N, 1)
    g3 = gathered.reshape(B, 1, D)
    grid = (B // bb,)
    return pl.pallas_call(
        _merge_body,
        grid=grid,
        in_specs=[
            pl.BlockSpec((bb, N, 1), lambda i: (i, 0, 0)),
            pl.BlockSpec((bb, N, D), lambda i: (i, 0, 0)),
            pl.BlockSpec((bb, 1, D), lambda i: (i, 0, 0)),
        ],
        out_specs=pl.BlockSpec((bb, N, D), lambda i: (i, 0, 0)),
        out_shape=jax.ShapeDtypeStruct((B, N, D), jnp.float32),
    )(tok3, embedded_text, g3)


def kernel(tokenized_text, embedded_text, name, params):
    params2d = params.reshape(NUM_NAMES, D)
    gathered = _sc_gather()(name, params2d)
    return _merge(tokenized_text, embedded_text, gathered, bb=8)


# speedup vs baseline: 1.0603x; 1.0603x over previous
import functools

import jax
import jax.numpy as jnp
from jax import lax
from jax.experimental import pallas as pl
from jax.experimental.pallas import tpu as pltpu
from jax.experimental.pallas import tpu_sc as plsc

B, N, D = 1024, 77, 768
NUM_NAMES = 1000
PLACEHOLDER_TOKEN = 265

_NC, _NS = 2, 16
_NW = _NC * _NS
_B_PER_W = B // _NW


def _sc_gather_body(name_hbm, params_hbm, out_hbm, idx_v, rows_v, sem):
    wid = lax.axis_index("s") * _NC + lax.axis_index("c")
    base = wid * _B_PER_W
    pltpu.sync_copy(name_hbm.at[pl.ds(base, _B_PER_W)], idx_v)
    pltpu.async_copy(params_hbm.at[idx_v], rows_v, sem).wait()
    pltpu.sync_copy(rows_v, out_hbm.at[pl.ds(base, _B_PER_W)])


@functools.cache
def _sc_gather():
    return pl.kernel(
        _sc_gather_body,
        out_type=jax.ShapeDtypeStruct((B, D), jnp.float32),
        mesh=plsc.VectorSubcoreMesh(core_axis_name="c", subcore_axis_name="s"),
        scratch_types=[
            pltpu.VMEM((_B_PER_W,), jnp.int32),
            pltpu.VMEM((_B_PER_W, D), jnp.float32),
            pltpu.SemaphoreType.DMA,
        ],
    )


def _merge_body(bb, tok_ref, emb_ref, g_ref, out_ref):
    for r in range(bb):
        m = tok_ref[r, :] == PLACEHOLDER_TOKEN
        col = jnp.sum(jnp.where(m, lax.iota(jnp.int32, N), 0))
        row_iota = lax.broadcasted_iota(jnp.int32, (N, D), 0)
        out_ref[r] = jnp.where(
            row_iota == col, g_ref[pl.ds(r, 1), :], emb_ref[r]
        )


def _merge(tokenized_text, embedded_text, gathered, bb):
    grid = (B // bb,)
    return pl.pallas_call(
        functools.partial(_merge_body, bb),
        grid=grid,
        in_specs=[
            pl.BlockSpec((bb, N), lambda i: (i, 0)),
            pl.BlockSpec((bb, N, D), lambda i: (i, 0, 0)),
            pl.BlockSpec((bb, D), lambda i: (i, 0)),
        ],
        out_specs=pl.BlockSpec((bb, N, D), lambda i: (i, 0, 0)),
        out_shape=jax.ShapeDtypeStruct((B, N, D), jnp.float32),
    )(tokenized_text, embedded_text, gathered)


def kernel(tokenized_text, embedded_text, name, params):
    params2d = params.reshape(NUM_NAMES, D)
    gathered = _sc_gather()(name, params2d)
    return _merge(tokenized_text, embedded_text, gathered, bb=8)


# speedup vs baseline: 1.0967x; 1.0344x over previous
import functools

import jax
import jax.numpy as jnp
from jax import lax
from jax.experimental import pallas as pl
from jax.experimental.pallas import tpu as pltpu
from jax.experimental.pallas import tpu_sc as plsc

B, N, D = 1024, 77, 768
NUM_NAMES = 1000
PLACEHOLDER_TOKEN = 265

_NC, _NS = 2, 16
_NW = _NC * _NS
_B_PER_W = B // _NW


def _sc_gather_body(name_hbm, params_hbm, out_hbm, idx_v, rows_v, sem):
    wid = lax.axis_index("s") * _NC + lax.axis_index("c")
    base = wid * _B_PER_W
    pltpu.sync_copy(name_hbm.at[pl.ds(base, _B_PER_W)], idx_v)
    pltpu.async_copy(params_hbm.at[idx_v], rows_v, sem).wait()
    pltpu.sync_copy(rows_v, out_hbm.at[pl.ds(base, _B_PER_W)])


@functools.cache
def _sc_gather():
    return pl.kernel(
        _sc_gather_body,
        out_type=jax.ShapeDtypeStruct((B, D), jnp.float32),
        mesh=plsc.VectorSubcoreMesh(core_axis_name="c", subcore_axis_name="s"),
        scratch_types=[
            pltpu.VMEM((_B_PER_W,), jnp.int32),
            pltpu.VMEM((_B_PER_W, D), jnp.float32),
            pltpu.SemaphoreType.DMA,
        ],
    )


def _merge_body(bb, tok_ref, emb_ref, g_ref, out_ref):
    for r in range(bb):
        m = tok_ref[r, :] == PLACEHOLDER_TOKEN
        col = jnp.sum(jnp.where(m, lax.iota(jnp.int32, N), 0))
        row_iota = lax.broadcasted_iota(jnp.int32, (N, D), 0)
        out_ref[r] = jnp.where(
            row_iota == col, g_ref[pl.ds(r, 1), :], emb_ref[r]
        )


def _merge(tokenized_text, embedded_text, gathered, bb):
    grid = (B // bb,)
    return pl.pallas_call(
        functools.partial(_merge_body, bb),
        grid=grid,
        in_specs=[
            pl.BlockSpec((bb, N), lambda i: (i, 0)),
            pl.BlockSpec((bb, N, D), lambda i: (i, 0, 0)),
            pl.BlockSpec((bb, D), lambda i: (i, 0)),
        ],
        out_specs=pl.BlockSpec((bb, N, D), lambda i: (i, 0, 0)),
        out_shape=jax.ShapeDtypeStruct((B, N, D), jnp.float32),
        compiler_params=pltpu.CompilerParams(
            dimension_semantics=("arbitrary",),
        ),
    )(tokenized_text, embedded_text, gathered)


def kernel(tokenized_text, embedded_text, name, params):
    params2d = params.reshape(NUM_NAMES, D)
    gathered = _sc_gather()(name, params2d)
    return _merge(tokenized_text, embedded_text, gathered, bb=32)
